# baseline (speedup 1.0000x reference)
import functools

import jax
import jax.numpy as jnp
from jax import lax
from jax.experimental import pallas as pl
from jax.experimental.pallas import tpu as pltpu
from jax.experimental.pallas import tpu_sc as plsc

L = 32768
H = 1024
K = 4
EPS = 1e-12

NC = 2
NS = 16
LANES = 16
NW = NC * NS
RPW = L // NW
RB = 32
NCHUNK = RPW // RB
NV = H // LANES


_GDN = lax.GatherDimensionNumbers(
    offset_dims=(), collapsed_slice_dims=(0,), start_index_map=(0,))


def _perm(v, idx):
    return lax.gather(v, idx[:, None], dimension_numbers=_GDN,
                      slice_sizes=(1,),
                      mode=lax.GatherScatterMode.PROMISE_IN_BOUNDS)


def _hsum16(v, lanes):
    for d in (8, 4, 2, 1):
        v = v + _perm(v, lanes ^ d)
    return v


def _rsqrt_vec(x):
    i = plsc.bitcast(x, jnp.int32)
    i = jnp.full((LANES,), 0x5F3759DF, jnp.int32) - (i >> 1)
    y = plsc.bitcast(i, jnp.float32)
    for _ in range(3):
        y = y * (1.5 - 0.5 * x * y * y)
    return y


def _sc_body(in_hbm, tab_hbm, gam_hbm, bet_hbm, idx_hbm, out_hbm,
             idx_v, in_v, pos_v, gam_v, bet_v, sem_a, sem_b):
    wid = lax.axis_index("s") * NC + lax.axis_index("c")
    base = wid * RPW
    pltpu.sync_copy(gam_hbm, gam_v)
    pltpu.sync_copy(bet_hbm, bet_v)

    def chunk_body(c, carry):
        r0 = base + c * RB
        pltpu.sync_copy(idx_hbm.at[pl.ds(r0, RB)], idx_v)
        cp_in = pltpu.async_copy(in_hbm.at[pl.ds(r0, RB)], in_v, sem_a)
        cp_pos = pltpu.async_copy(tab_hbm.at[idx_v], pos_v, sem_b)
        cp_in.wait()
        cp_pos.wait()

        lanes = lax.iota(jnp.int32, LANES)

        def row_body(r, carry2):
            def p1(j, acc):
                s, s2 = acc
                v = in_v[r, pl.ds(j * LANES, LANES)] + pos_v[r, pl.ds(j * LANES, LANES)]
                in_v[r, pl.ds(j * LANES, LANES)] = v
                return s + v, s2 + v * v
            zero = jnp.zeros((LANES,), jnp.float32)
            s, s2 = lax.fori_loop(0, NV, p1, (zero, zero))
            mu = _hsum16(s, lanes) * (1.0 / H)
            var = _hsum16(s2, lanes) * (1.0 / H) - mu * mu
            rs = _rsqrt_vec(var + EPS)

            def p2(j, _):
                sl = pl.ds(j * LANES, LANES)
                v = (in_v[r, sl] - mu) * rs
                in_v[r, sl] = v * gam_v[sl] + bet_v[sl]
                return 0
            lax.fori_loop(0, NV, p2, 0)
            return carry2
        lax.fori_loop(0, RB, row_body, 0)
        pltpu.sync_copy(in_v, out_hbm.at[pl.ds(r0, RB)])
        return carry
    lax.fori_loop(0, NCHUNK, chunk_body, 0)


_sc_kernel = functools.partial(
    pl.kernel,
    mesh=plsc.VectorSubcoreMesh(core_axis_name="c", subcore_axis_name="s"),
    out_type=jax.ShapeDtypeStruct((L, H), jnp.float32),
    compiler_params=pltpu.CompilerParams(needs_layout_passes=False),
    scratch_types=[
        pltpu.VMEM((RB,), jnp.int32),
        pltpu.VMEM((RB, H), jnp.float32),
        pltpu.VMEM((RB, H), jnp.float32),
        pltpu.VMEM((H,), jnp.float32),
        pltpu.VMEM((H,), jnp.float32),
        pltpu.SemaphoreType.DMA,
        pltpu.SemaphoreType.DMA,
    ],
)(_sc_body)


@jax.jit
def kernel(input_enc, emb_table, ln_gamma, ln_beta, embedding_index):
    return _sc_kernel(input_enc, emb_table, ln_gamma, ln_beta,
                      embedding_index.astype(jnp.int32))


# speedup vs baseline: 1.6850x; 1.6850x over previous
import functools

import jax
import jax.numpy as jnp
from jax import lax
from jax.experimental import pallas as pl
from jax.experimental.pallas import tpu as pltpu
from jax.experimental.pallas import tpu_sc as plsc

L = 32768
H = 1024
K = 4
EPS = 1e-12

NC = 2
NS = 16
LANES = 16
NW = NC * NS
RPW = L // NW
RB = 16
NCHUNK = RPW // RB
NV = H // LANES

_GDN = lax.GatherDimensionNumbers(
    offset_dims=(), collapsed_slice_dims=(0,), start_index_map=(0,))


def _perm(v, idx):
    return lax.gather(v, idx[:, None], dimension_numbers=_GDN,
                      slice_sizes=(1,),
                      mode=lax.GatherScatterMode.PROMISE_IN_BOUNDS)


def _hsum16(v, lanes):
    for d in (8, 4, 2, 1):
        v = v + _perm(v, lanes ^ d)
    return v


def _rsqrt_vec(x):
    i = plsc.bitcast(x, jnp.int32)
    i = jnp.full((LANES,), 0x5F3759DF, jnp.int32) - (i >> 1)
    y = plsc.bitcast(i, jnp.float32)
    for _ in range(3):
        y = y * (1.5 - 0.5 * x * y * y)
    return y


def _sc_body(in_hbm, tab_hbm, gam_hbm, bet_hbm, idx_hbm, out_hbm,
             idx_all, in_b, pos_b, gam_v, bet_v, sem_in, sem_pos, sem_out):
    wid = lax.axis_index("s") * NC + lax.axis_index("c")
    base = wid * RPW
    pltpu.sync_copy(gam_hbm, gam_v)
    pltpu.sync_copy(bet_hbm, bet_v)
    pltpu.sync_copy(idx_hbm.at[pl.ds(base, RPW)], idx_all)

    def fill(b, c):
        r0 = base + c * RB
        pltpu.async_copy(in_hbm.at[pl.ds(r0, RB)], in_b[b], sem_in[b])
        pltpu.async_copy(tab_hbm.at[idx_all.at[pl.ds(c * RB, RB)]],
                         pos_b[b], sem_pos[b])

    def compute(b):
        iv = in_b[b]
        pv = pos_b[b]
        lanes = lax.iota(jnp.int32, LANES)

        def row_body(r, carry):
            acc = [jnp.zeros((LANES,), jnp.float32) for _ in range(4)]
            acc2 = [jnp.zeros((LANES,), jnp.float32) for _ in range(4)]
            for j in range(NV):
                sl = pl.ds(j * LANES, LANES)
                v = iv[r, sl] + pv[r, sl]
                iv[r, sl] = v
                acc[j % 4] = acc[j % 4] + v
                acc2[j % 4] = acc2[j % 4] + v * v
            s = (acc[0] + acc[1]) + (acc[2] + acc[3])
            s2 = (acc2[0] + acc2[1]) + (acc2[2] + acc2[3])
            mu = _hsum16(s, lanes) * (1.0 / H)
            var = _hsum16(s2, lanes) * (1.0 / H) - mu * mu
            rs = _rsqrt_vec(var + EPS)
            for j in range(NV):
                sl = pl.ds(j * LANES, LANES)
                iv[r, sl] = (iv[r, sl] - mu) * rs * gam_v[sl] + bet_v[sl]
            return carry
        lax.fori_loop(0, RB, row_body, 0)

    fill(0, 0)

    def outer(c2, carry):
        for b in (0, 1):
            c = 2 * c2 + b
            r0 = base + c * RB
            nb = 1 - b

            @pl.when(c + 1 < NCHUNK)
            def _prefetch():
                @pl.when(c >= 1)
                def _drain_out():
                    pltpu.make_async_copy(
                        in_b[nb], out_hbm.at[pl.ds(r0, RB)], sem_out[nb]
                    ).wait()
                fill(nb, c + 1)

            pltpu.make_async_copy(
                in_hbm.at[pl.ds(r0, RB)], in_b[b], sem_in[b]).wait()
            pltpu.make_async_copy(
                tab_hbm.at[idx_all.at[pl.ds(c * RB, RB)]], pos_b[b],
                sem_pos[b]).wait()
            compute(b)
            pltpu.async_copy(in_b[b], out_hbm.at[pl.ds(r0, RB)], sem_out[b])
        return carry

    lax.fori_loop(0, NCHUNK // 2, outer, 0)
    for b in (0, 1):
        pltpu.make_async_copy(
            in_b[b], out_hbm.at[pl.ds(base, RB)], sem_out[b]).wait()


def _sc_entry(body):
    return pl.kernel(
        body,
        mesh=plsc.VectorSubcoreMesh(core_axis_name="c", subcore_axis_name="s"),
        out_type=jax.ShapeDtypeStruct((L, H), jnp.float32),
        compiler_params=pltpu.CompilerParams(needs_layout_passes=False),
        scratch_types=[
            pltpu.VMEM((RPW,), jnp.int32),
            [pltpu.VMEM((RB, H), jnp.float32) for _ in range(2)],
            [pltpu.VMEM((RB, H), jnp.float32) for _ in range(2)],
            pltpu.VMEM((H,), jnp.float32),
            pltpu.VMEM((H,), jnp.float32),
            [pltpu.SemaphoreType.DMA for _ in range(2)],
            [pltpu.SemaphoreType.DMA for _ in range(2)],
            [pltpu.SemaphoreType.DMA for _ in range(2)],
        ],
    )


_sc_kernel = _sc_entry(_sc_body)


@jax.jit
def kernel(input_enc, emb_table, ln_gamma, ln_beta, embedding_index):
    return _sc_kernel(input_enc, emb_table, ln_gamma, ln_beta,
                      embedding_index.astype(jnp.int32))


# speedup vs baseline: 12.1540x; 7.2130x over previous
import jax
import jax.numpy as jnp
from jax.experimental import pallas as pl

L = 32768
H = 1024
K = 4
EPS = 1e-12
BLOCK = 2048


def _ln_body(idx_ref, in_ref, tab_ref, gam_ref, bet_ref, out_ref):
    idx = idx_ref[0, 0, :]
    x = in_ref[...]
    tab = tab_ref[...]
    ks = jax.lax.broadcasted_iota(jnp.int32, (BLOCK, K), 1)
    onehot = (idx[:, None] == ks).astype(jnp.float32)
    pos = jnp.dot(onehot, tab, preferred_element_type=jnp.float32)
    x = x + pos
    mu = jnp.mean(x, axis=-1, keepdims=True)
    var = jnp.mean(x * x, axis=-1, keepdims=True) - mu * mu
    y = (x - mu) * jax.lax.rsqrt(var + EPS)
    out_ref[...] = y * gam_ref[...] + bet_ref[...]


@jax.jit
def kernel(input_enc, emb_table, ln_gamma, ln_beta, embedding_index):
    nb = L // BLOCK
    idx3 = embedding_index.astype(jnp.int32).reshape(nb, 1, BLOCK)
    gam = ln_gamma.reshape(1, H)
    bet = ln_beta.reshape(1, H)
    return pl.pallas_call(
        _ln_body,
        grid=(nb,),
        in_specs=[
            pl.BlockSpec((1, 1, BLOCK), lambda i: (i, 0, 0)),
            pl.BlockSpec((BLOCK, H), lambda i: (i, 0)),
            pl.BlockSpec((K, H), lambda i: (0, 0)),
            pl.BlockSpec((1, H), lambda i: (0, 0)),
            pl.BlockSpec((1, H), lambda i: (0, 0)),
        ],
        out_specs=pl.BlockSpec((BLOCK, H), lambda i: (i, 0)),
        out_shape=jax.ShapeDtypeStruct((L, H), jnp.float32),
    )(idx3, input_enc, emb_table, gam, bet)
